# baseline (speedup 1.0000x reference)
import functools

import jax
import jax.numpy as jnp
from jax import lax
from jax.experimental import pallas as pl
from jax.experimental.pallas import tpu as pltpu
from jax.experimental.pallas import tpu_sc as plsc

_B = 1024
_D = 64
_N = 100000
_C = 2000
_CP = 2048
_TEMP = 0.05
_TOPP = 0.1
_NC = 2
_NS = 16
_NW = _NC * _NS
_CH = 125
_NH = _N // 2
_NCHH = _NH // _CH
_CPT = _NCHH // _NW
_XTRA = _NCHH % _NW
_LPAD = 416
_CW = 16
_RB = 1024
_NRB = _B // _RB


def _sc_stage(features, labels3, z64):
    mesh = plsc.VectorSubcoreMesh(
        core_axis_name="c", subcore_axis_name="s",
        num_cores=_NC, num_subcores=_NS)

    @functools.partial(
        pl.kernel,
        out_type=(
            jax.ShapeDtypeStruct((_NC, _CP, _D), jnp.float32),
            jax.ShapeDtypeStruct((_NC, _NS, _CP), jnp.float32),
        ),
        mesh=mesh,
        compiler_params=pltpu.CompilerParams(use_tc_tiling_on_sc=False,
                                             needs_layout_passes=False),
        scratch_types=[
            pltpu.VMEM((_CH, _D), jnp.float32),
            pltpu.VMEM((_CH, _D), jnp.float32),
            pltpu.VMEM((_CPT + 1, _CH), jnp.int32),
            pltpu.VMEM((_CP,), jnp.float32),
            pltpu.VMEM_SHARED((_CP, _D), jnp.float32),
            pltpu.SemaphoreType.DMA,
            pltpu.SemaphoreType.DMA,
        ],
    )
    def k(feat_hbm, lab2_hbm, z64_hbm,
          acc_out, cnt_out,
          rows0_v, rows1_v, labs_v, hist_v, acc_sh, sem0, sem1):
        c = lax.axis_index("c")
        s = lax.axis_index("s")
        wid = c * _NS + s
        gstart = wid * _CPT + jnp.minimum(wid, _XTRA)
        bufs = (rows0_v, rows1_v)
        sems = (sem0, sem1)

        @pl.when(s == 0)
        def _init():
            pltpu.sync_copy(z64_hbm, acc_sh)

        pltpu.sync_copy(lab2_hbm.at[pl.ds(gstart, _CPT + 1)], labs_v)

        zer16 = jnp.zeros((16,), jnp.float32)

        def hz(kk, carry):
            hist_v[pl.ds(kk * 16, 16)] = zer16
            return carry

        lax.fori_loop(0, _CP // 16, hz, 0)
        plsc.subcore_barrier()

        def dma_start(j, b):
            off = jnp.minimum((gstart + j) * _CH, _NH - _CH)
            pltpu.async_copy(feat_hbm.at[pl.ds(off, _CH)], bufs[b], sems[b])

        def dma_wait(b):
            pltpu.make_async_copy(feat_hbm.at[pl.ds(0, _CH)],
                                  bufs[b], sems[b]).wait()

        one16 = jnp.ones((16,), jnp.float32)
        lane = lax.iota(jnp.int32, 16)
        tailmask = lane >= (16 - (_CH - (_CH // 16) * 16))

        def scat(j, b):
            pltpu.sync_copy(bufs[b], acc_sh.at[labs_v.at[j]], add=True)
            for kk in range(_CH // 16):
                lab16 = labs_v[j, pl.ds(kk * 16, 16)]
                plsc.addupdate_scatter(hist_v, [lab16], one16)
            lab16 = labs_v[j, pl.ds(_CH - 16, 16)]
            plsc.addupdate_scatter(hist_v, [lab16], one16, mask=tailmask)

        dma_start(0, 0)

        def pipe(i, carry):
            j = i * 2
            dma_wait(0)
            dma_start(j + 1, 1)
            scat(j, 0)
            dma_wait(1)
            dma_start(j + 2, 0)
            scat(j + 1, 1)
            return carry

        lax.fori_loop(0, _CPT // 2, pipe, 0)
        dma_wait(0)

        @pl.when(wid < _XTRA)
        def _extra():
            scat(_CPT, 0)

        plsc.subcore_barrier()

        @pl.when(s == 0)
        def _writeout():
            pltpu.sync_copy(acc_sh, acc_out.at[c])

        pltpu.sync_copy(hist_v, cnt_out.at[c, s])

    return k(features, labels3, z64)


def _sc_targets(labels, indexes):
    mesh = plsc.VectorSubcoreMesh(
        core_axis_name="c", subcore_axis_name="s",
        num_cores=_NC, num_subcores=_NS)

    @functools.partial(
        pl.kernel,
        out_type=jax.ShapeDtypeStruct((_B,), jnp.int32),
        mesh=mesh,
        compiler_params=pltpu.CompilerParams(use_tc_tiling_on_sc=False,
                                             needs_layout_passes=False),
        scratch_types=[
            pltpu.VMEM((_N,), jnp.int32),
            pltpu.VMEM((_B,), jnp.int32),
            pltpu.VMEM((_B // 2,), jnp.int32),
        ],
    )
    def k(lab_hbm, idx_hbm, tgt_out, laball_v, idxs_v, tgt_v):
        c = lax.axis_index("c")
        s = lax.axis_index("s")

        @pl.when(s == 0)
        def _gather():
            half = _B // 2
            pltpu.sync_copy(lab_hbm, laball_v)
            pltpu.sync_copy(idx_hbm, idxs_v)

            def gath(i, carry):
                idx16 = idxs_v[pl.ds(c * half + i * 16, 16)]
                tgt_v[pl.ds(i * 16, 16)] = plsc.load_gather(laball_v, [idx16])
                return carry

            lax.fori_loop(0, half // 16, gath, 0)
            pltpu.sync_copy(tgt_v, tgt_out.at[pl.ds(c * half, half)])

    return k(labels, indexes)


def _tc_body(res_ref, acca_ref, accb_ref, cnta_ref, cntb_ref, tgt_ref,
             out_ref, en_ref):
    i = pl.program_id(0)

    @pl.when(i == 0)
    def _zero():
        out_ref[0, 0] = 0.0

    x = res_ref[...]
    x = x / jnp.maximum(jnp.sqrt(jnp.sum(x * x, axis=1, keepdims=True)), 1e-12)
    csum = (acca_ref[0] + acca_ref[1]) + (accb_ref[0] + accb_ref[1])
    cnt3 = (jnp.sum(cnta_ref[...], axis=1, keepdims=True)
            + jnp.sum(cntb_ref[...], axis=1, keepdims=True))
    cnt = cnt3[0] + cnt3[1]
    sim = lax.dot_general(x, csum, (((1,), (1,)), ((), ())),
                          preferred_element_type=jnp.float32)
    sim = sim / (_TEMP * jnp.maximum(cnt, 1.0))
    cidx = lax.broadcasted_iota(jnp.int32, (1, _CP), 1)
    maskc = jnp.logical_and(cnt > 0.0, cidx < _C)
    e = jnp.where(maskc, jnp.exp(sim), 0.0)
    col = lax.broadcasted_iota(jnp.int32, (_RB, _CP), 1)
    is_pos = col == tgt_ref[...]
    pos = jnp.sum(jnp.where(is_pos, e, 0.0), axis=1, keepdims=True)
    en = jnp.where(is_pos, 0.0, e)
    en_ref[...] = en
    total = jnp.sum(en, axis=1, keepdims=True)
    p_total = _TOPP * total

    hi0 = lax.bitcast_convert_type(jnp.max(en, axis=1, keepdims=True),
                                   jnp.int32) + 1
    lo0 = jnp.full_like(hi0, 822251815)

    def bis(_, carry):
        lo, hi = carry
        mid = lo + (hi - lo) // 2
        v = en_ref[...]
        vb = lax.bitcast_convert_type(v, jnp.int32)
        g = jnp.sum(jnp.where(vb >= mid, v, 0.0), axis=1, keepdims=True)
        pred = g > p_total
        return jnp.where(pred, mid, lo), jnp.where(pred, hi, mid)

    lo, hi = lax.fori_loop(0, 29, bis, (lo0, hi0))

    en2 = en_ref[...]
    enb = lax.bitcast_convert_type(en2, jnp.int32)
    geA = enb >= hi
    cumA = jnp.sum(jnp.where(geA, en2, 0.0), axis=1, keepdims=True)
    cntA = jnp.sum(jnp.where(geA, 1.0, 0.0), axis=1, keepdims=True)
    e_next = jnp.max(jnp.where(geA, 0.0, en2), axis=1, keepdims=True)
    cumB = cumA + e_next
    chooseA = jnp.logical_and(cntA >= 1.0,
                              jnp.abs(cumA - p_total) <= jnp.abs(cumB - p_total))
    s_thr = jnp.where(chooseA, cumA, cumB)

    val = pos / (pos + s_thr + 1e-6)
    logp = jnp.log(val + 1e-6)
    out_ref[0, 0] += jnp.sum(logp)

    @pl.when(i == _NRB - 1)
    def _fin():
        out_ref[0, 0] = -out_ref[0, 0] / _B


def _tc_stage(results, acc_a, acc_b, cnt_a, cnt_b, targets2):
    out = pl.pallas_call(
        _tc_body,
        grid=(_NRB,),
        in_specs=[
            pl.BlockSpec((_RB, _D), lambda i: (i, 0)),
            pl.BlockSpec((_NC, _CP, _D), lambda i: (0, 0, 0)),
            pl.BlockSpec((_NC, _CP, _D), lambda i: (0, 0, 0)),
            pl.BlockSpec((_NC, _NS, _CP), lambda i: (0, 0, 0)),
            pl.BlockSpec((_NC, _NS, _CP), lambda i: (0, 0, 0)),
            pl.BlockSpec((_RB, 1), lambda i: (i, 0)),
        ],
        out_specs=pl.BlockSpec((1, 1), lambda i: (0, 0),
                               memory_space=pltpu.SMEM),
        out_shape=jax.ShapeDtypeStruct((1, 1), jnp.float32),
        scratch_shapes=[pltpu.VMEM((_RB, _CP), jnp.float32)],
    )(results, acc_a, acc_b, cnt_a, cnt_b, targets2)
    return out


def kernel(results, indexes, features, labels):
    lab2 = labels.reshape(2 * _NCHH, _CH)
    pad = jnp.full((_LPAD - _NCHH, _CH), _CP - 1, jnp.int32)
    lab2_a = jnp.concatenate([lab2[:_NCHH], pad], axis=0)
    lab2_b = jnp.concatenate([lab2[_NCHH:], pad], axis=0)
    z64 = jnp.zeros((_CP, _D), jnp.float32)
    acc_a, cnt_a = _sc_stage(features[:_NH], lab2_a, z64)
    acc_b, cnt_b = _sc_stage(features[_NH:], lab2_b, z64)
    tgt = _sc_targets(labels, indexes.astype(jnp.int32))
    targets2 = tgt.reshape(_B, 1)
    out = _tc_stage(results, acc_a, acc_b, cnt_a, cnt_b, targets2)
    return out.reshape(())


# speedup vs baseline: 1.1948x; 1.1948x over previous
import functools

import jax
import jax.numpy as jnp
from jax import lax
from jax.experimental import pallas as pl
from jax.experimental.pallas import tpu as pltpu
from jax.experimental.pallas import tpu_sc as plsc

_B = 1024
_D = 64
_N = 100000
_C = 2000
_CP = 2048
_TEMP = 0.05
_TOPP = 0.1
_NC = 2
_NS = 16
_NW = _NC * _NS
_RPT = _N // _NW
_CH = 125
_NCH = _RPT // _CH
_CW = 16
_RB = 1024
_NRB = _B // _RB


def _sc_stage(features, labels3, z64):
    mesh = plsc.VectorSubcoreMesh(
        core_axis_name="c", subcore_axis_name="s",
        num_cores=_NC, num_subcores=_NS)

    @functools.partial(
        pl.kernel,
        out_type=(
            jax.ShapeDtypeStruct((_NC, _CP, _D), jnp.float32),
            jax.ShapeDtypeStruct((_NC, _NS, _CP), jnp.float32),
        ),
        mesh=mesh,
        compiler_params=pltpu.CompilerParams(use_tc_tiling_on_sc=False,
                                             needs_layout_passes=False),
        scratch_types=[
            pltpu.VMEM((_CH, _D), jnp.float32),
            pltpu.VMEM((_CH, _D), jnp.float32),
            pltpu.VMEM((_NCH, _CH), jnp.int32),
            pltpu.VMEM((_CP,), jnp.float32),
            pltpu.VMEM_SHARED((_CP, _D), jnp.float32),
            pltpu.SemaphoreType.DMA,
            pltpu.SemaphoreType.DMA,
        ],
    )
    def k(feat_hbm, lab3_hbm, z64_hbm,
          acc_out, cnt_out,
          rows0_v, rows1_v, labs_v, hist_v, acc_sh, sem0, sem1):
        c = lax.axis_index("c")
        s = lax.axis_index("s")
        wid = c * _NS + s
        base = wid * _RPT
        bufs = (rows0_v, rows1_v)
        sems = (sem0, sem1)

        @pl.when(s == 0)
        def _init():
            pltpu.sync_copy(z64_hbm, acc_sh)

        pltpu.sync_copy(lab3_hbm.at[wid], labs_v)

        zer16 = jnp.zeros((16,), jnp.float32)

        def hz(kk, carry):
            hist_v[pl.ds(kk * 16, 16)] = zer16
            return carry

        lax.fori_loop(0, _CP // 16, hz, 0)
        plsc.subcore_barrier()

        def dma_start(j, b):
            pltpu.async_copy(feat_hbm.at[pl.ds(base + j * _CH, _CH)],
                             bufs[b], sems[b])

        def dma_wait(b):
            pltpu.make_async_copy(feat_hbm.at[pl.ds(0, _CH)],
                                  bufs[b], sems[b]).wait()

        one16 = jnp.ones((16,), jnp.float32)
        lane = lax.iota(jnp.int32, 16)
        tailmask = lane >= (16 - (_CH - (_CH // 16) * 16))

        def scat(j, b):
            pltpu.sync_copy(bufs[b], acc_sh.at[labs_v.at[j]], add=True)
            for kk in range(_CH // 16):
                lab16 = labs_v[j, pl.ds(kk * 16, 16)]
                plsc.addupdate_scatter(hist_v, [lab16], one16)
            lab16 = labs_v[j, pl.ds(_CH - 16, 16)]
            plsc.addupdate_scatter(hist_v, [lab16], one16, mask=tailmask)

        dma_start(0, 0)

        def pipe(i, carry):
            j = i * 2
            dma_wait(0)
            dma_start(j + 1, 1)
            scat(j, 0)
            dma_wait(1)
            dma_start(j + 2, 0)
            scat(j + 1, 1)
            return carry

        lax.fori_loop(0, (_NCH - 1) // 2, pipe, 0)
        dma_wait(0)
        scat(_NCH - 1, 0)
        plsc.subcore_barrier()

        @pl.when(s == 0)
        def _writeout():
            pltpu.sync_copy(acc_sh, acc_out.at[c])

        pltpu.sync_copy(hist_v, cnt_out.at[c, s])

    return k(features, labels3, z64)


def _sc_targets(labels, indexes):
    mesh = plsc.VectorSubcoreMesh(
        core_axis_name="c", subcore_axis_name="s",
        num_cores=_NC, num_subcores=_NS)

    @functools.partial(
        pl.kernel,
        out_type=jax.ShapeDtypeStruct((_B,), jnp.int32),
        mesh=mesh,
        compiler_params=pltpu.CompilerParams(use_tc_tiling_on_sc=False,
                                             needs_layout_passes=False),
        scratch_types=[
            pltpu.VMEM((_N,), jnp.int32),
            pltpu.VMEM((_B,), jnp.int32),
            pltpu.VMEM((_B // 2,), jnp.int32),
        ],
    )
    def k(lab_hbm, idx_hbm, tgt_out, laball_v, idxs_v, tgt_v):
        c = lax.axis_index("c")
        s = lax.axis_index("s")

        @pl.when(s == 0)
        def _gather():
            half = _B // 2
            pltpu.sync_copy(lab_hbm, laball_v)
            pltpu.sync_copy(idx_hbm, idxs_v)

            def gath(i, carry):
                idx16 = idxs_v[pl.ds(c * half + i * 16, 16)]
                tgt_v[pl.ds(i * 16, 16)] = plsc.load_gather(laball_v, [idx16])
                return carry

            lax.fori_loop(0, half // 16, gath, 0)
            pltpu.sync_copy(tgt_v, tgt_out.at[pl.ds(c * half, half)])

    return k(labels, indexes)


def _tc_body(res_ref, acc_ref, cnt_ref, tgt_ref, out_ref, en_ref):
    i = pl.program_id(0)

    @pl.when(i == 0)
    def _zero():
        out_ref[0, 0] = 0.0

    x = res_ref[...]
    x = x / jnp.maximum(jnp.sqrt(jnp.sum(x * x, axis=1, keepdims=True)), 1e-12)
    csum = acc_ref[0] + acc_ref[1]
    cnt3 = jnp.sum(cnt_ref[...], axis=1, keepdims=True)
    cnt = cnt3[0] + cnt3[1]
    sim = lax.dot_general(x, csum, (((1,), (1,)), ((), ())),
                          preferred_element_type=jnp.float32)
    sim = sim / (_TEMP * jnp.maximum(cnt, 1.0))
    cidx = lax.broadcasted_iota(jnp.int32, (1, _CP), 1)
    maskc = jnp.logical_and(cnt > 0.0, cidx < _C)
    e = jnp.where(maskc, jnp.exp(sim), 0.0)
    col = lax.broadcasted_iota(jnp.int32, (_RB, _CP), 1)
    is_pos = col == tgt_ref[...]
    pos = jnp.sum(jnp.where(is_pos, e, 0.0), axis=1, keepdims=True)
    en = jnp.where(is_pos, 0.0, e)
    en_ref[...] = en
    total = jnp.sum(en, axis=1, keepdims=True)
    p_total = _TOPP * total

    hi0 = lax.bitcast_convert_type(jnp.max(en, axis=1, keepdims=True),
                                   jnp.int32) + 1
    lo0 = jnp.full_like(hi0, 822251815)

    def bis(_, carry):
        lo, hi = carry
        mid = lo + (hi - lo) // 2
        v = en_ref[...]
        vb = lax.bitcast_convert_type(v, jnp.int32)
        g = jnp.sum(jnp.where(vb >= mid, v, 0.0), axis=1, keepdims=True)
        pred = g > p_total
        return jnp.where(pred, mid, lo), jnp.where(pred, hi, mid)

    lo, hi = lax.fori_loop(0, 29, bis, (lo0, hi0))

    en2 = en_ref[...]
    enb = lax.bitcast_convert_type(en2, jnp.int32)
    geA = enb >= hi
    cumA = jnp.sum(jnp.where(geA, en2, 0.0), axis=1, keepdims=True)
    cntA = jnp.sum(jnp.where(geA, 1.0, 0.0), axis=1, keepdims=True)
    e_next = jnp.max(jnp.where(geA, 0.0, en2), axis=1, keepdims=True)
    cumB = cumA + e_next
    chooseA = jnp.logical_and(cntA >= 1.0,
                              jnp.abs(cumA - p_total) <= jnp.abs(cumB - p_total))
    s_thr = jnp.where(chooseA, cumA, cumB)

    val = pos / (pos + s_thr + 1e-6)
    logp = jnp.log(val + 1e-6)
    out_ref[0, 0] += jnp.sum(logp)

    @pl.when(i == _NRB - 1)
    def _fin():
        out_ref[0, 0] = -out_ref[0, 0] / _B


def _tc_stage(results, acc, cnt2, targets2):
    out = pl.pallas_call(
        _tc_body,
        grid=(_NRB,),
        in_specs=[
            pl.BlockSpec((_RB, _D), lambda i: (i, 0)),
            pl.BlockSpec((_NC, _CP, _D), lambda i: (0, 0, 0)),
            pl.BlockSpec((_NC, _NS, _CP), lambda i: (0, 0, 0)),
            pl.BlockSpec((_RB, 1), lambda i: (i, 0)),
        ],
        out_specs=pl.BlockSpec((1, 1), lambda i: (0, 0),
                               memory_space=pltpu.SMEM),
        out_shape=jax.ShapeDtypeStruct((1, 1), jnp.float32),
        scratch_shapes=[pltpu.VMEM((_RB, _CP), jnp.float32)],
    )(results, acc, cnt2, targets2)
    return out


def kernel(results, indexes, features, labels):
    labels3 = labels.reshape(_NW, _NCH, _CH)
    z64 = jnp.zeros((_CP, _D), jnp.float32)
    acc, cnt = _sc_stage(features, labels3, z64)
    tgt = _sc_targets(labels, indexes.astype(jnp.int32))
    targets2 = tgt.reshape(_B, 1)
    out = _tc_stage(results, acc, cnt, targets2)
    return out.reshape(())
